# baseline (speedup 1.0000x reference)
import functools

import jax
import jax.numpy as jnp
from jax import lax
from jax.experimental import pallas as pl
from jax.experimental.pallas import tpu as pltpu
from jax.experimental.pallas import tpu_sc as plsc

K = 1024
CB = 4
D = 64
N = 16384
RB = 512
HALVES = 2
NH = N // HALVES
NBLK = NH // RB
BETA = 0.25

NC = 2
NS = 16
NW = NC * NS
TOT = CB * NH
ROWS_W = TOT // NW
CH = 512
NCH = ROWS_W // CH
GCH = 128
NG = CH // GCH


def _make_tc(off, with_init, interpret=False):
    def body(*refs):
        if with_init:
            x_ref, emb_ref, lin_ref, idx_ref, loss_ref, wsq_ref = refs
        else:
            x_ref, emb_ref, idx_ref, loss_ref, wsq_ref = refs
        g = pl.program_id(0)

        @pl.when(g == 0)
        def _():
            if with_init:
                loss_ref[:, :] = lin_ref[:, :]
            else:
                loss_ref[:, :] = jnp.zeros((1, 1), jnp.float32)
            for i in range(CB):
                w = emb_ref[i]
                wsq_ref[i:i + 1, :] = jnp.sum(w * w, axis=1)[None, :]

        acc = jnp.zeros((), jnp.float32)
        iota = lax.broadcasted_iota(jnp.int32, (RB, K), 1)
        for i in range(CB):
            a = x_ref[:, i:i + D]
            w2 = emb_ref[i] + emb_ref[i]
            c2 = lax.dot_general(a, w2, (((1,), (1,)), ((), ())),
                                 preferred_element_type=jnp.float32)
            xsq = jnp.sum(a * a, axis=1, keepdims=True)
            dist = (xsq + wsq_ref[i:i + 1, :]) - c2
            md = jnp.min(dist, axis=1, keepdims=True)
            arg = jnp.min(jnp.where(dist == md, iota, K), axis=1, keepdims=True)
            idx_ref[:, i:i + 1] = arg + i * K
            acc = acc + jnp.sum(md)

        loss_ref[:, :] = loss_ref[:, :] + (acc * ((1.0 + BETA) / (N * D))).reshape(1, 1)

    in_specs = [
        pl.BlockSpec((RB, CB * D), lambda g: (g + off, 0)),
        pl.BlockSpec((CB, K, D), lambda g: (0, 0, 0)),
    ]
    if with_init:
        in_specs.append(pl.BlockSpec((1, 1), lambda g: (0, 0)))
    return pl.pallas_call(
        body,
        grid=(NBLK,),
        in_specs=in_specs,
        out_specs=[
            pl.BlockSpec((RB, CB), lambda g: (g, 0)),
            pl.BlockSpec((1, 1), lambda g: (0, 0)),
        ],
        out_shape=[
            jax.ShapeDtypeStruct((NH, CB), jnp.int32),
            jax.ShapeDtypeStruct((1, 1), jnp.float32),
        ],
        scratch_shapes=[pltpu.VMEM((CB, K), jnp.float32)],
        interpret=interpret,
    )


def _sc_gather_body(table_hbm, idx_hbm, out_hbm, idx_v, rows_a, rows_b, sem_a,
                    sem_b):
    wid = lax.axis_index("s") * NC + lax.axis_index("c")
    base = wid * ROWS_W
    pltpu.sync_copy(idx_hbm.at[pl.ds(base, ROWS_W)], idx_v)
    bufs = (rows_a, rows_b)
    sems = (sem_a, sem_b)

    def fire(ci, buf, sem):
        cps = []
        for gj in range(NG):
            cps.append(pltpu.async_copy(
                table_hbm.at[idx_v.at[pl.ds(ci * CH + gj * GCH, GCH)]],
                buf.at[pl.ds(gj * GCH, GCH)], sem))
        return cps

    def drain_store(ci, cps, buf):
        for cp in cps:
            cp.wait()
        pltpu.sync_copy(buf, out_hbm.at[pl.ds(base + ci * CH, CH)])

    cps = fire(0, bufs[0], sems[0])
    for ci in range(1, NCH):
        nxt = fire(ci, bufs[ci % 2], sems[ci % 2])
        drain_store(ci - 1, cps, bufs[(ci - 1) % 2])
        cps = nxt
    drain_store(NCH - 1, cps, bufs[(NCH - 1) % 2])


@functools.cache
def _sc_gather():
    return pl.kernel(
        _sc_gather_body,
        out_type=jax.ShapeDtypeStruct((TOT, D), jnp.float32),
        mesh=plsc.VectorSubcoreMesh(core_axis_name="c", subcore_axis_name="s"),
        scratch_types=[
            pltpu.VMEM((ROWS_W,), jnp.int32),
            pltpu.VMEM((CH, D), jnp.float32),
            pltpu.VMEM((CH, D), jnp.float32),
            pltpu.SemaphoreType.DMA,
            pltpu.SemaphoreType.DMA,
        ],
        compiler_params=pltpu.CompilerParams(use_tc_tiling_on_sc=False),
    )


def kernel(latents, emb):
    B, C, H, W = latents.shape
    xt = jnp.transpose(latents, (0, 2, 3, 1)).reshape(N, C)
    table = emb.reshape(CB * K, D)
    idx0, l0 = _make_tc(0, False)(xt, emb)
    q0 = _sc_gather()(table, idx0.reshape(-1))
    idx1, l1 = _make_tc(NBLK, True)(xt, emb, l0)
    q1 = _sc_gather()(table, idx1.reshape(-1))
    qs = [q.reshape(B // HALVES, H, W, CB, D) for q in (q0, q1)]
    quant = (jnp.concatenate(qs, axis=0).transpose(0, 3, 4, 1, 2)
             .reshape(B, C, H, W))
    return quant, l1[0, 0]
